# baseline (speedup 1.0000x reference)
import jax
import jax.numpy as jnp
from jax import lax
from jax.experimental import pallas as pl
from jax.experimental.pallas import tpu as pltpu

_NSPLIT = 4


def _mlp_t_kernel(*refs):
    x_refs = refs[:_NSPLIT]
    w1_ref, b1_ref, w2t_ref, b2_ref, o_ref = refs[_NSPLIT:]
    w1b = (w1_ref[...] * 0.5).astype(jnp.bfloat16)
    b1h = 0.5 * b1_ref[...].T
    w2t = w2t_ref[...]
    w2tb = (w2t * 0.5).astype(jnp.bfloat16)
    c2 = 0.5 * jnp.sum(w2t, axis=1, keepdims=True) + b2_ref[...].T
    part = x_refs[0].shape[1]
    for k, x_ref in enumerate(x_refs):
        xt = x_ref[...].astype(jnp.bfloat16)
        z = lax.dot_general(w1b, xt, (((0,), (0,)), ((), ())),
                            preferred_element_type=jnp.float32)
        t = jnp.tanh(z + b1h)
        y = jnp.dot(w2tb, t.astype(jnp.bfloat16),
                    preferred_element_type=jnp.float32)
        o_ref[:, k * part:(k + 1) * part] = (y + c2).astype(o_ref.dtype)


def kernel(x, w1, b1, w2, b2):
    B, D_in = x.shape
    H = w1.shape[1]
    D_out = w2.shape[1]

    xt = x.T
    w2t = w2.T
    B0 = B

    tile_n = 65536
    while B % tile_n != 0:
        tile_n //= 2
    if tile_n < _NSPLIT:
        pad = _NSPLIT - B % _NSPLIT if B % _NSPLIT else 0
        xt = jnp.pad(xt, ((0, 0), (0, pad)))
        B = B + pad
        tile_n = _NSPLIT
    part = tile_n // _NSPLIT
    grid = (B // tile_n,)

    def _xspec(k):
        return pl.BlockSpec((D_in, part), lambda i, k=k: (0, _NSPLIT * i + k))

    out_t = pl.pallas_call(
        _mlp_t_kernel,
        out_shape=jax.ShapeDtypeStruct((D_out, B), x.dtype),
        grid_spec=pl.GridSpec(
            grid=grid,
            in_specs=[_xspec(k) for k in range(_NSPLIT)] + [
                pl.BlockSpec((D_in, H), lambda i: (0, 0)),
                pl.BlockSpec((1, H), lambda i: (0, 0)),
                pl.BlockSpec((D_out, H), lambda i: (0, 0)),
                pl.BlockSpec((1, D_out), lambda i: (0, 0)),
            ],
            out_specs=pl.BlockSpec((D_out, tile_n), lambda i: (0, i)),
        ),
        compiler_params=pltpu.CompilerParams(
            dimension_semantics=("parallel",),
            vmem_limit_bytes=64 * 1024 * 1024,
        ),
    )(*([xt] * _NSPLIT + [w1, b1, w2t, b2]))

    if B != B0:
        out_t = out_t[:, :B0]
    return out_t.T


# speedup vs baseline: 1.0255x; 1.0255x over previous
import jax
import jax.numpy as jnp
from jax import lax
from jax.experimental import pallas as pl
from jax.experimental.pallas import tpu as pltpu


def _mlp_t_kernel(xt_ref, w1_ref, b1_ref, w2t_ref, b2_ref, o_ref):
    w1b = (w1_ref[...] * 0.5).astype(jnp.bfloat16)
    b1h = 0.5 * b1_ref[...].T
    w2t = w2t_ref[...]
    w2tb = (w2t * 0.5).astype(jnp.bfloat16)
    c2 = 0.5 * jnp.sum(w2t, axis=1, keepdims=True) + b2_ref[...].T
    xt = xt_ref[...].astype(jnp.bfloat16)
    z = lax.dot_general(w1b, xt, (((0,), (0,)), ((), ())),
                        preferred_element_type=jnp.float32)
    t = jnp.tanh(z + b1h)
    y = jnp.dot(w2tb, t.astype(jnp.bfloat16),
                preferred_element_type=jnp.float32)
    o_ref[...] = (y + c2).astype(o_ref.dtype)


def kernel(x, w1, b1, w2, b2):
    B, D_in = x.shape
    H = w1.shape[1]
    D_out = w2.shape[1]

    xt = x.T
    w2t = w2.T

    tile_n = 65536
    while B % tile_n != 0 and tile_n > 1:
        tile_n //= 2
    grid = (B // tile_n,)

    out_t = pl.pallas_call(
        _mlp_t_kernel,
        out_shape=jax.ShapeDtypeStruct((D_out, B), x.dtype),
        grid_spec=pl.GridSpec(
            grid=grid,
            in_specs=[
                pl.BlockSpec((D_in, tile_n), lambda i: (0, i)),
                pl.BlockSpec((D_in, H), lambda i: (0, 0)),
                pl.BlockSpec((1, H), lambda i: (0, 0)),
                pl.BlockSpec((D_out, H), lambda i: (0, 0)),
                pl.BlockSpec((1, D_out), lambda i: (0, 0)),
            ],
            out_specs=pl.BlockSpec((D_out, tile_n), lambda i: (0, i)),
        ),
        compiler_params=pltpu.CompilerParams(
            dimension_semantics=("parallel",),
            vmem_limit_bytes=64 * 1024 * 1024,
        ),
    )(xt, w1, b1, w2t, b2)

    return out_t.T
